# Initial kernel scaffold; baseline (speedup 1.0000x reference)
import jax
import jax.numpy as jnp
from jax.experimental import pallas as pl


def kernel(inputs):
    raise NotImplementedError("write your pallas kernel here")


# speedup vs baseline: 6.3009x; 6.3009x over previous
import functools
import math

import jax
import jax.numpy as jnp
from jax.experimental import pallas as pl

_NUM_UNITS = 1024


def _pe_tile_kernel(out_ref, *, n, t_tile, num_units):
    t = pl.program_id(0)
    shape = (t_tile, num_units)
    pos_i = jax.lax.broadcasted_iota(jnp.int32, shape, 0) + t * t_tile
    pos = pos_i.astype(jnp.float32)
    col_i = jax.lax.broadcasted_iota(jnp.int32, shape, 1)
    col_f = col_i.astype(jnp.float32)
    inv_freq = jnp.exp(col_f * jnp.float32(-2.0 * math.log(10000.0) / num_units))
    angle = pos * inv_freq
    parity = (col_i & 1).astype(jnp.float32)
    val = jnp.sin(angle + parity * jnp.float32(math.pi / 2.0))
    val = jnp.where(pos == 0.0, jnp.float32(0.0), val) * jnp.float32(num_units**0.5)
    out_ref[...] = jnp.broadcast_to(val[None], (n, t_tile, num_units))


def kernel(inputs):
    n, t_total = inputs.shape
    num_units = _NUM_UNITS
    t_tile = 256
    grid = (t_total // t_tile,)
    out = pl.pallas_call(
        functools.partial(_pe_tile_kernel, n=n, t_tile=t_tile, num_units=num_units),
        grid=grid,
        out_specs=pl.BlockSpec((n, t_tile, num_units), lambda t: (0, t, 0)),
        out_shape=jax.ShapeDtypeStruct((n, t_total, num_units), jnp.float32),
    )()
    return out


# speedup vs baseline: 14.9810x; 2.3776x over previous
import functools
import math

import jax
import jax.numpy as jnp
from jax.experimental import pallas as pl
from jax.experimental.pallas import tpu as pltpu

_NUM_UNITS = 1024
_K = 64


def _pe_tile_kernel(out_ref, sa_ref, ca_ref, sb_ref, cb_ref, *, n, t_tile, k,
                    num_units, n_hi):
    t = pl.program_id(0)
    half_pi = jnp.float32(math.pi / 2.0)
    neg_log_rate = jnp.float32(-2.0 * math.log(10000.0) / num_units)
    scale = jnp.float32(num_units**0.5)

    @pl.when(t == 0)
    def _init_tables():
        col_b = jax.lax.broadcasted_iota(jnp.int32, (k, num_units), 1)
        w_b = jnp.exp(col_b.astype(jnp.float32) * neg_log_rate)
        lo = jax.lax.broadcasted_iota(jnp.int32, (k, num_units), 0)
        parity = (col_b & 1).astype(jnp.float32)
        b = lo.astype(jnp.float32) * w_b + parity * half_pi
        sb_ref[...] = jnp.sin(b) * scale
        cb_ref[...] = jnp.sin(b + half_pi) * scale
        col_a = jax.lax.broadcasted_iota(jnp.int32, (n_hi, num_units), 1)
        w_a = jnp.exp(col_a.astype(jnp.float32) * neg_log_rate)
        hi = jax.lax.broadcasted_iota(jnp.int32, (n_hi, num_units), 0)
        a = (hi * k).astype(jnp.float32) * w_a
        sa_ref[...] = jnp.sin(a)
        ca_ref[...] = jnp.sin(a + half_pi)

    chunks = t_tile // k
    for j in range(chunks):
        hi_idx = t * chunks + j
        a_s = sa_ref[pl.ds(hi_idx, 1), :]
        a_c = ca_ref[pl.ds(hi_idx, 1), :]
        val = a_s * cb_ref[...] + a_c * sb_ref[...]
        out_ref[:, j * k:(j + 1) * k, :] = jnp.broadcast_to(
            val[None], (n, k, num_units))

    @pl.when(t == 0)
    def _zero_row0():
        out_ref[:, 0:1, :] = jnp.zeros((n, 1, num_units), jnp.float32)


def kernel(inputs):
    n, t_total = inputs.shape
    num_units = _NUM_UNITS
    t_tile = 256
    k = _K
    n_hi = t_total // k
    grid = (t_total // t_tile,)
    out = pl.pallas_call(
        functools.partial(_pe_tile_kernel, n=n, t_tile=t_tile, k=k,
                          num_units=num_units, n_hi=n_hi),
        grid=grid,
        out_specs=pl.BlockSpec((n, t_tile, num_units), lambda t: (0, t, 0)),
        out_shape=jax.ShapeDtypeStruct((n, t_total, num_units), jnp.float32),
        scratch_shapes=[
            pltpu.VMEM((n_hi, num_units), jnp.float32),
            pltpu.VMEM((n_hi, num_units), jnp.float32),
            pltpu.VMEM((k, num_units), jnp.float32),
            pltpu.VMEM((k, num_units), jnp.float32),
        ],
    )()
    return out
